# baseline (speedup 1.0000x reference)
import functools

import jax
import jax.numpy as jnp
from jax import lax
from jax.experimental import pallas as pl
from jax.experimental.pallas import tpu as pltpu
from jax.experimental.pallas import tpu_sc as plsc

_CHUNK = 128
_BK = 768


@functools.partial(jax.jit, static_argnames=("V", "D", "NC", "NW"))
def _sc_row_major(v_t, *, V, D, NC, NW):
    BK = _BK
    n_full = V // BK
    main = (n_full // NW) * NW
    tail0 = n_full * BK
    tail_rest = V - tail0
    TP = -(-tail_rest // 128) * 128 if tail_rest else 0

    mesh = plsc.VectorSubcoreMesh(core_axis_name="c", subcore_axis_name="s")

    @functools.partial(
        pl.kernel,
        out_type=jax.ShapeDtypeStruct((V * D,), jnp.float32),
        mesh=mesh,
        compiler_params=pltpu.CompilerParams(needs_layout_passes=False),
        scratch_types=[
            pltpu.VMEM((D, BK), jnp.float32),
            pltpu.VMEM((D, BK), jnp.float32),
            pltpu.VMEM((BK * (D + 1),), jnp.float32),
            pltpu.VMEM((BK * D,), jnp.float32),
            pltpu.VMEM((BK * D,), jnp.float32),
            pltpu.SemaphoreType.DMA,
            pltpu.SemaphoreType.DMA,
            pltpu.SemaphoreType.DMA,
            pltpu.SemaphoreType.DMA,
        ],
    )
    def k1(v_t_hbm, v_tail_hbm, v_out_hbm, si0, si1, sp, so0, so1,
           isem0, isem1, osem0, osem1):
        wid = lax.axis_index("s") * NC + lax.axis_index("c")
        iota = lax.iota(jnp.int32, D)
        idxc = [iota * (D + 1) + d for d in range(D)]
        sis, sos = (si0, si1), (so0, so1)
        isems, osems = (isem0, isem1), (osem0, osem1)

        def start_in(tab_hbm, src_col, width, p):
            return [
                pltpu.async_copy(
                    tab_hbm.at[:, pl.ds(src_col, width)],
                    sis[p].at[:, pl.ds(0, width)], isems[p]),
            ]

        def shuffle(width, p):
            si, so = sis[p], sos[p]

            def g_body(g, carry):
                off = g * 16
                base = g * (16 * (D + 1))
                for d in range(D):
                    x = si[d, pl.ds(off, 16)]
                    plsc.store_scatter(sp, [idxc[d] + base], x)
                return carry
            lax.fori_loop(0, width // 16, g_body, 0, unroll=4)

            def c_body(g, carry):
                for j in range(16):
                    v = g * 16 + j
                    so[pl.ds(v * D, D)] = sp[pl.ds(v * (D + 1), D)]
                return carry
            lax.fori_loop(0, width // 16, c_body, 0, unroll=4)

        def start_out(out_hbm, out_col, out_width, p):
            return pltpu.async_copy(
                sos[p].at[pl.ds(0, out_width * D)],
                out_hbm.at[pl.ds(out_col * D, out_width * D)], osems[p])

        def wait_in(tab_hbm, p):
            pltpu.make_async_copy(
                tab_hbm.at[:, pl.ds(0, BK)],
                sis[p].at[:, pl.ds(0, BK)], isems[p]).wait()

        def wait_out(out_hbm, p):
            pltpu.make_async_copy(
                sos[p].at[pl.ds(0, BK * D)],
                out_hbm.at[pl.ds(0, BK * D)], osems[p]).wait()

        n_main = main // NW
        n_pairs = n_main // 2
        for tab_hbm, tail_hbm, out_hbm in (
                (v_t_hbm, v_tail_hbm, v_out_hbm),):
            def col(k):
                return (wid + k * NW) * BK

            start_in(tab_hbm, col(0), BK, 0)

            def pair_body(kk, carry, tab=tab_hbm, out=out_hbm):
                ka = 2 * kk
                start_in(tab, col(ka + 1), BK, 1)
                wait_in(tab, 0)

                @pl.when(kk > 0)
                def _():
                    wait_out(out, 0)
                shuffle(BK, 0)
                start_out(out, col(ka), BK, 0)

                @pl.when(kk < n_pairs - 1)
                def _():
                    start_in(tab, col(ka + 2), BK, 0)
                wait_in(tab, 1)

                @pl.when(kk > 0)
                def _():
                    wait_out(out, 1)
                shuffle(BK, 1)
                start_out(out, col(ka + 1), BK, 1)
                return carry
            lax.fori_loop(0, n_pairs, pair_body, 0)
            wait_out(out_hbm, 0)
            wait_out(out_hbm, 1)

            @pl.when(wid < n_full - main)
            def _():
                c = (main + wid) * BK
                for cp in start_in(tab_hbm, c, BK, 0):
                    cp.wait()
                shuffle(BK, 0)
                start_out(out_hbm, c, BK, 0).wait()
            if tail_rest:
                @pl.when(wid == NW - 1)
                def _():
                    for cp in start_in(tail_hbm, 0, TP, 1):
                        cp.wait()
                    shuffle(TP, 1)
                    start_out(out_hbm, tail0, tail_rest, 1).wait()

    if tail_rest:
        v_tail = jnp.pad(v_t[:, tail0:], ((0, 0), (0, TP - tail_rest)))
    else:
        v_tail = jnp.zeros((D, 128), jnp.float32)
    return k1(v_t, v_tail)


@functools.partial(jax.jit, static_argnames=("B", "V", "D", "N", "NC", "NW"))
def _sc_partials(v_pos, neg_idx, v_rm, *, B, V, D, N, NC, NW):
    BW = B // NW
    C = _CHUNK
    n_chunks = BW // C

    mesh = plsc.VectorSubcoreMesh(core_axis_name="c", subcore_axis_name="s")

    @functools.partial(
        pl.kernel,
        out_type=(
            jax.ShapeDtypeStruct((B, D), jnp.float32),
            jax.ShapeDtypeStruct((B * D,), jnp.float32),
        ),
        mesh=mesh,
        compiler_params=pltpu.CompilerParams(use_tc_tiling_on_sc=False),
        scratch_types=[
            pltpu.VMEM((BW,), jnp.int32),
            pltpu.VMEM((BW * N,), jnp.int32),
            pltpu.VMEM((C, D), jnp.float32),
            pltpu.VMEM((C * N, D), jnp.float32),
            pltpu.VMEM((C * D,), jnp.float32),
            pltpu.SemaphoreType.DMA,
        ],
    )
    def sc_kernel(v_pos_hbm, neg_hbm, v_w_hbm,
                  v_out_hbm, ns_out_hbm,
                  v_idx_v, neg_idx_v, v_rows_v,
                  neg_rows_v, out_ns_v, sem):
        wid = lax.axis_index("s") * NC + lax.axis_index("c")

        pltpu.sync_copy(v_pos_hbm.at[pl.ds(wid * BW, BW)], v_idx_v)
        pltpu.sync_copy(neg_hbm.at[pl.ds(wid * BW * N, BW * N)], neg_idx_v)

        def chunk_body(ci, carry):
            base = (wid * BW + ci * C) * D
            cps = [
                pltpu.async_copy(
                    v_w_hbm.at[v_idx_v.at[pl.ds(ci * C, C)]], v_rows_v, sem),
            ]
            for j in range(N):
                cps.append(pltpu.async_copy(
                    v_w_hbm.at[neg_idx_v.at[pl.ds((ci * N + j) * C, C)]],
                    neg_rows_v.at[pl.ds(j * C, C)], sem))
            for cp in cps:
                cp.wait()

            def b_body(b, carry2):
                acc = neg_rows_v[b * N]
                for nn in range(1, N):
                    acc = acc + neg_rows_v[b * N + nn]
                out_ns_v[pl.ds(b * D, D)] = acc
                return carry2
            lax.fori_loop(0, C, b_body, 0)

            cpv = pltpu.async_copy(
                v_rows_v, v_out_hbm.at[pl.ds(wid * BW + ci * C, C)], sem)
            pltpu.sync_copy(out_ns_v, ns_out_hbm.at[pl.ds(base, C * D)])
            cpv.wait()
            return carry
        lax.fori_loop(0, n_chunks, chunk_body, 0)

    return sc_kernel(v_pos, neg_idx, v_rm)


def _tc_loss_body(u_ref, v_ref, ns_ref, out_ref, *, D, inv_b):
    xu = u_ref[...]
    xp = xu * v_ref[...]
    xn = xu * ns_ref[...]
    lanes = xp.shape[-1]
    g = lanes // D
    i0 = lax.broadcasted_iota(jnp.int32, (lanes, g), 0)
    i1 = lax.broadcasted_iota(jnp.int32, (lanes, g), 1)
    s_mat = (i0 // D == i1).astype(jnp.float32)
    ps = jnp.dot(xp, s_mat, preferred_element_type=jnp.float32)
    ns = jnp.dot(xn, s_mat, preferred_element_type=jnp.float32)

    def log_sigmoid(x):
        return jnp.minimum(x, 0.0) - jnp.log1p(jnp.exp(-jnp.abs(x)))

    loss = log_sigmoid(ps) + log_sigmoid(ns)
    out_ref[0, 0] = -jnp.sum(loss) * inv_b


def kernel(u_pos, v_pos, v_neg, batch_size, u_weight, v_weight):
    B = u_pos.shape[0]
    V, D = u_weight.shape
    N = v_neg.shape[-1]
    info = plsc.get_sparse_core_info()
    NC, NS = info.num_cores, info.num_subcores
    NW = NC * NS

    v_lin = _sc_row_major(v_weight.T, V=V, D=D, NC=NC, NW=NW)

    neg_idx = v_neg.reshape(-1).astype(jnp.int32)
    embed_v, neg_sum = _sc_partials(
        v_pos.astype(jnp.int32), neg_idx, v_lin.reshape(V, D),
        B=B, V=V, D=D, N=N, NC=NC, NW=NW)

    embed_u = jnp.take(u_weight, u_pos, axis=0)

    lanes = 128
    rows = B * D // lanes
    u2d = embed_u.reshape(rows, lanes)
    v2d = embed_v.reshape(rows, lanes)
    ns2d = neg_sum.reshape(rows, lanes)

    out = pl.pallas_call(
        functools.partial(_tc_loss_body, D=D, inv_b=1.0 / B),
        out_shape=jax.ShapeDtypeStruct((1, 1), jnp.float32),
        out_specs=pl.BlockSpec(memory_space=pltpu.SMEM),
    )(u2d, v2d, ns2d)
    return out.reshape(())


# speedup vs baseline: 1.5863x; 1.5863x over previous
import functools

import jax
import jax.numpy as jnp
from jax import lax
from jax.experimental import pallas as pl
from jax.experimental.pallas import tpu as pltpu
from jax.experimental.pallas import tpu_sc as plsc

_CHUNK = 128
_BK = 1024


@functools.partial(jax.jit, static_argnames=("V", "D", "NC", "NW"))
def _sc_row_major(v_t, *, V, D, NC, NW):
    BK = _BK
    n_full = V // BK
    main = (n_full // NW) * NW
    tail0 = n_full * BK
    tail_rest = V - tail0
    TP = -(-tail_rest // 128) * 128 if tail_rest else 0

    mesh = plsc.VectorSubcoreMesh(core_axis_name="c", subcore_axis_name="s")

    @functools.partial(
        pl.kernel,
        out_type=jax.ShapeDtypeStruct((V * D,), jnp.float32),
        mesh=mesh,
        compiler_params=pltpu.CompilerParams(needs_layout_passes=False),
        scratch_types=[
            pltpu.VMEM((D, BK), jnp.float32),
            pltpu.VMEM((D, BK), jnp.float32),
            pltpu.VMEM((BK * D,), jnp.float32),
            pltpu.VMEM((BK * D,), jnp.float32),
            pltpu.SemaphoreType.DMA,
            pltpu.SemaphoreType.DMA,
            pltpu.SemaphoreType.DMA,
            pltpu.SemaphoreType.DMA,
        ],
    )
    def k1(v_t_hbm, v_tail_hbm, v_out_hbm, si0, si1, so0, so1,
           isem0, isem1, osem0, osem1):
        wid = lax.axis_index("s") * NC + lax.axis_index("c")
        iota = lax.iota(jnp.int32, D)
        idxc = [iota * D + d for d in range(D)]
        sis, sos = (si0, si1), (so0, so1)
        isems, osems = (isem0, isem1), (osem0, osem1)

        def start_in(tab_hbm, src_col, width, p):
            return [
                pltpu.async_copy(
                    tab_hbm.at[:, pl.ds(src_col, width)],
                    sis[p].at[:, pl.ds(0, width)], isems[p]),
            ]

        def shuffle(width, p):
            si, so = sis[p], sos[p]

            def g_body(g, carry):
                off = g * 16
                base = g * (16 * D)
                for d in range(D):
                    x = si[d, pl.ds(off, 16)]
                    plsc.store_scatter(so, [idxc[d] + base], x)
                return carry
            lax.fori_loop(0, width // 16, g_body, 0, unroll=4)

        def start_out(out_hbm, out_col, out_width, p):
            return pltpu.async_copy(
                sos[p].at[pl.ds(0, out_width * D)],
                out_hbm.at[pl.ds(out_col * D, out_width * D)], osems[p])

        def wait_in(tab_hbm, p):
            pltpu.make_async_copy(
                tab_hbm.at[:, pl.ds(0, BK)],
                sis[p].at[:, pl.ds(0, BK)], isems[p]).wait()

        def wait_out(out_hbm, p):
            pltpu.make_async_copy(
                sos[p].at[pl.ds(0, BK * D)],
                out_hbm.at[pl.ds(0, BK * D)], osems[p]).wait()

        n_main = main // NW
        n_pairs = n_main // 2
        for tab_hbm, tail_hbm, out_hbm in (
                (v_t_hbm, v_tail_hbm, v_out_hbm),):
            def col(k):
                return (wid + k * NW) * BK

            start_in(tab_hbm, col(0), BK, 0)

            def pair_body(kk, carry, tab=tab_hbm, out=out_hbm):
                ka = 2 * kk
                start_in(tab, col(ka + 1), BK, 1)
                wait_in(tab, 0)

                @pl.when(kk > 0)
                def _():
                    wait_out(out, 0)
                shuffle(BK, 0)
                start_out(out, col(ka), BK, 0)

                @pl.when(kk < n_pairs - 1)
                def _():
                    start_in(tab, col(ka + 2), BK, 0)
                wait_in(tab, 1)

                @pl.when(kk > 0)
                def _():
                    wait_out(out, 1)
                shuffle(BK, 1)
                start_out(out, col(ka + 1), BK, 1)
                return carry
            lax.fori_loop(0, n_pairs, pair_body, 0)
            wait_out(out_hbm, 0)
            wait_out(out_hbm, 1)

            @pl.when(wid < n_full - main)
            def _():
                c = (main + wid) * BK
                for cp in start_in(tab_hbm, c, BK, 0):
                    cp.wait()
                shuffle(BK, 0)
                start_out(out_hbm, c, BK, 0).wait()
            if tail_rest:
                @pl.when(wid == NW - 1)
                def _():
                    for cp in start_in(tail_hbm, 0, TP, 1):
                        cp.wait()
                    shuffle(TP, 1)
                    start_out(out_hbm, tail0, tail_rest, 1).wait()

    if tail_rest:
        v_tail = jnp.pad(v_t[:, tail0:], ((0, 0), (0, TP - tail_rest)))
    else:
        v_tail = jnp.zeros((D, 128), jnp.float32)
    return k1(v_t, v_tail)


@functools.partial(jax.jit, static_argnames=("B", "V", "D", "N", "NC", "NW"))
def _sc_partials(v_pos, neg_idx, v_rm, *, B, V, D, N, NC, NW):
    BW = B // NW
    C = _CHUNK
    n_chunks = BW // C

    mesh = plsc.VectorSubcoreMesh(core_axis_name="c", subcore_axis_name="s")

    @functools.partial(
        pl.kernel,
        out_type=(
            jax.ShapeDtypeStruct((B, D), jnp.float32),
            jax.ShapeDtypeStruct((B * D,), jnp.float32),
        ),
        mesh=mesh,
        compiler_params=pltpu.CompilerParams(use_tc_tiling_on_sc=False),
        scratch_types=[
            pltpu.VMEM((BW,), jnp.int32),
            pltpu.VMEM((BW * N,), jnp.int32),
            pltpu.VMEM((C, D), jnp.float32),
            pltpu.VMEM((C * N, D), jnp.float32),
            pltpu.VMEM((C * D,), jnp.float32),
            pltpu.SemaphoreType.DMA,
        ],
    )
    def sc_kernel(v_pos_hbm, neg_hbm, v_w_hbm,
                  v_out_hbm, ns_out_hbm,
                  v_idx_v, neg_idx_v, v_rows_v,
                  neg_rows_v, out_ns_v, sem):
        wid = lax.axis_index("s") * NC + lax.axis_index("c")

        pltpu.sync_copy(v_pos_hbm.at[pl.ds(wid * BW, BW)], v_idx_v)
        pltpu.sync_copy(neg_hbm.at[pl.ds(wid * BW * N, BW * N)], neg_idx_v)

        def chunk_body(ci, carry):
            base = (wid * BW + ci * C) * D
            cps = [
                pltpu.async_copy(
                    v_w_hbm.at[v_idx_v.at[pl.ds(ci * C, C)]], v_rows_v, sem),
            ]
            for j in range(N):
                cps.append(pltpu.async_copy(
                    v_w_hbm.at[neg_idx_v.at[pl.ds((ci * N + j) * C, C)]],
                    neg_rows_v.at[pl.ds(j * C, C)], sem))
            for cp in cps:
                cp.wait()

            def b_body(b, carry2):
                rows = [neg_rows_v[b * N + nn] for nn in range(N)]
                while len(rows) > 1:
                    rows = [rows[i] + rows[i + 1] for i in range(0, len(rows) - 1, 2)] + (
                        [rows[-1]] if len(rows) % 2 else [])
                out_ns_v[pl.ds(b * D, D)] = rows[0]
                return carry2
            lax.fori_loop(0, C, b_body, 0, unroll=2)

            cpv = pltpu.async_copy(
                v_rows_v, v_out_hbm.at[pl.ds(wid * BW + ci * C, C)], sem)
            pltpu.sync_copy(out_ns_v, ns_out_hbm.at[pl.ds(base, C * D)])
            cpv.wait()
            return carry
        lax.fori_loop(0, n_chunks, chunk_body, 0)

    return sc_kernel(v_pos, neg_idx, v_rm)


def _tc_loss_body(u_ref, v_ref, ns_ref, out_ref, *, D, inv_b):
    xu = u_ref[...]
    xp = xu * v_ref[...]
    xn = xu * ns_ref[...]
    lanes = xp.shape[-1]
    g = lanes // D
    i0 = lax.broadcasted_iota(jnp.int32, (lanes, g), 0)
    i1 = lax.broadcasted_iota(jnp.int32, (lanes, g), 1)
    s_mat = (i0 // D == i1).astype(jnp.float32)
    ps = jnp.dot(xp, s_mat, preferred_element_type=jnp.float32)
    ns = jnp.dot(xn, s_mat, preferred_element_type=jnp.float32)

    def log_sigmoid(x):
        return jnp.minimum(x, 0.0) - jnp.log1p(jnp.exp(-jnp.abs(x)))

    loss = log_sigmoid(ps) + log_sigmoid(ns)
    out_ref[0, 0] = -jnp.sum(loss) * inv_b


def kernel(u_pos, v_pos, v_neg, batch_size, u_weight, v_weight):
    B = u_pos.shape[0]
    V, D = u_weight.shape
    N = v_neg.shape[-1]
    info = plsc.get_sparse_core_info()
    NC, NS = info.num_cores, info.num_subcores
    NW = NC * NS

    v_lin = _sc_row_major(v_weight.T, V=V, D=D, NC=NC, NW=NW)

    neg_idx = v_neg.reshape(-1).astype(jnp.int32)
    embed_v, neg_sum = _sc_partials(
        v_pos.astype(jnp.int32), neg_idx, v_lin.reshape(V, D),
        B=B, V=V, D=D, N=N, NC=NC, NW=NW)

    embed_u = jnp.take(u_weight, u_pos, axis=0)

    lanes = 128
    rows = B * D // lanes
    u2d = embed_u.reshape(rows, lanes)
    v2d = embed_v.reshape(rows, lanes)
    ns2d = neg_sum.reshape(rows, lanes)

    out = pl.pallas_call(
        functools.partial(_tc_loss_body, D=D, inv_b=1.0 / B),
        out_shape=jax.ShapeDtypeStruct((1, 1), jnp.float32),
        out_specs=pl.BlockSpec(memory_space=pltpu.SMEM),
    )(u2d, v2d, ns2d)
    return out.reshape(())


# speedup vs baseline: 1.6475x; 1.0386x over previous
import functools

import jax
import jax.numpy as jnp
from jax import lax
from jax.experimental import pallas as pl
from jax.experimental.pallas import tpu as pltpu
from jax.experimental.pallas import tpu_sc as plsc

_CHUNK = 128
_BK = 1024


@functools.partial(jax.jit, static_argnames=("V", "D", "NC", "NW"))
def _sc_row_major(v_t, *, V, D, NC, NW):
    BK = _BK
    n_full = V // BK
    main = (n_full // NW) * NW
    tail0 = n_full * BK
    tail_rest = V - tail0
    TP = -(-tail_rest // 128) * 128 if tail_rest else 0

    mesh = plsc.VectorSubcoreMesh(core_axis_name="c", subcore_axis_name="s")

    @functools.partial(
        pl.kernel,
        out_type=jax.ShapeDtypeStruct((V * D,), jnp.float32),
        mesh=mesh,
        compiler_params=pltpu.CompilerParams(needs_layout_passes=False),
        scratch_types=[
            pltpu.VMEM((D, BK), jnp.float32),
            pltpu.VMEM((D, BK), jnp.float32),
            pltpu.VMEM((BK * D,), jnp.float32),
            pltpu.VMEM((BK * D,), jnp.float32),
            pltpu.SemaphoreType.DMA,
            pltpu.SemaphoreType.DMA,
            pltpu.SemaphoreType.DMA,
            pltpu.SemaphoreType.DMA,
        ],
    )
    def k1(v_t_hbm, v_tail_hbm, v_out_hbm, si0, si1, so0, so1,
           isem0, isem1, osem0, osem1):
        wid = lax.axis_index("s") * NC + lax.axis_index("c")
        iota = lax.iota(jnp.int32, D)
        idxc = [iota * D + d for d in range(D)]
        sis, sos = (si0, si1), (so0, so1)
        isems, osems = (isem0, isem1), (osem0, osem1)

        def start_in(tab_hbm, src_col, width, p):
            return [
                pltpu.async_copy(
                    tab_hbm.at[:, pl.ds(src_col, width)],
                    sis[p].at[:, pl.ds(0, width)], isems[p]),
            ]

        def shuffle(width, p):
            si, so = sis[p], sos[p]

            def g_body(g, carry):
                off = g * 16
                base = g * (16 * D)
                for d in range(D):
                    x = si[d, pl.ds(off, 16)]
                    plsc.store_scatter(so, [idxc[d] + base], x)
                return carry
            lax.fori_loop(0, width // 16, g_body, 0, unroll=4)

        def start_out(out_hbm, out_col, out_width, p):
            return pltpu.async_copy(
                sos[p].at[pl.ds(0, out_width * D)],
                out_hbm.at[pl.ds(out_col * D, out_width * D)], osems[p])

        def wait_in(tab_hbm, p):
            pltpu.make_async_copy(
                tab_hbm.at[:, pl.ds(0, BK)],
                sis[p].at[:, pl.ds(0, BK)], isems[p]).wait()

        def wait_out(out_hbm, p):
            pltpu.make_async_copy(
                sos[p].at[pl.ds(0, BK * D)],
                out_hbm.at[pl.ds(0, BK * D)], osems[p]).wait()

        n_main = main // NW
        n_pairs = n_main // 2
        for tab_hbm, tail_hbm, out_hbm in (
                (v_t_hbm, v_tail_hbm, v_out_hbm),):
            def col(k):
                return (wid + k * NW) * BK

            start_in(tab_hbm, col(0), BK, 0)

            def pair_body(kk, carry, tab=tab_hbm, out=out_hbm):
                ka = 2 * kk
                start_in(tab, col(ka + 1), BK, 1)
                wait_in(tab, 0)

                @pl.when(kk > 0)
                def _():
                    wait_out(out, 0)
                shuffle(BK, 0)
                start_out(out, col(ka), BK, 0)

                @pl.when(kk < n_pairs - 1)
                def _():
                    start_in(tab, col(ka + 2), BK, 0)
                wait_in(tab, 1)

                @pl.when(kk > 0)
                def _():
                    wait_out(out, 1)
                shuffle(BK, 1)
                start_out(out, col(ka + 1), BK, 1)
                return carry
            lax.fori_loop(0, n_pairs, pair_body, 0)
            wait_out(out_hbm, 0)
            wait_out(out_hbm, 1)

            @pl.when(wid < n_full - main)
            def _():
                c = (main + wid) * BK
                for cp in start_in(tab_hbm, c, BK, 0):
                    cp.wait()
                shuffle(BK, 0)
                start_out(out_hbm, c, BK, 0).wait()
            if tail_rest:
                @pl.when(wid == NW - 1)
                def _():
                    for cp in start_in(tail_hbm, 0, TP, 1):
                        cp.wait()
                    shuffle(TP, 1)
                    start_out(out_hbm, tail0, tail_rest, 1).wait()

    if tail_rest:
        v_tail = jnp.pad(v_t[:, tail0:], ((0, 0), (0, TP - tail_rest)))
    else:
        v_tail = jnp.zeros((D, 128), jnp.float32)
    return k1(v_t, v_tail)


@functools.partial(jax.jit, static_argnames=("B", "V", "D", "N", "NC", "NW"))
def _sc_partials(v_pos, neg_idx, v_rm, *, B, V, D, N, NC, NW):
    BW = B // NW
    C = 64
    n_chunks = BW // C

    mesh = plsc.VectorSubcoreMesh(core_axis_name="c", subcore_axis_name="s")

    @functools.partial(
        pl.kernel,
        out_type=(
            jax.ShapeDtypeStruct((B, D), jnp.float32),
            jax.ShapeDtypeStruct((B * D,), jnp.float32),
        ),
        mesh=mesh,
        compiler_params=pltpu.CompilerParams(use_tc_tiling_on_sc=False),
        scratch_types=[
            pltpu.VMEM((BW,), jnp.int32),
            pltpu.VMEM((BW * N,), jnp.int32),
            pltpu.VMEM((C, D), jnp.float32),
            pltpu.VMEM((C, D), jnp.float32),
            pltpu.VMEM((C * N, D), jnp.float32),
            pltpu.VMEM((C * N, D), jnp.float32),
            pltpu.VMEM((C * D,), jnp.float32),
            pltpu.VMEM((C * D,), jnp.float32),
            pltpu.SemaphoreType.DMA,
            pltpu.SemaphoreType.DMA,
            pltpu.SemaphoreType.DMA,
            pltpu.SemaphoreType.DMA,
        ],
    )
    def sc_kernel(v_pos_hbm, neg_hbm, v_w_hbm,
                  v_out_hbm, ns_out_hbm,
                  v_idx_v, neg_idx_v, vr0, vr1, nr0, nr1, ns0, ns1,
                  gsem0, gsem1, osem0, osem1):
        wid = lax.axis_index("s") * NC + lax.axis_index("c")
        vrs, nrs, nss = (vr0, vr1), (nr0, nr1), (ns0, ns1)
        gsems, osems = (gsem0, gsem1), (osem0, osem1)

        def wait_gathers(p):
            pltpu.make_async_copy(
                v_w_hbm.at[pl.ds(0, C)], vrs[p], gsems[p]).wait()
            pltpu.make_async_copy(
                v_w_hbm.at[pl.ds(0, N * C)], nrs[p], gsems[p]).wait()

        def wait_outs(p):
            pltpu.make_async_copy(
                vrs[p], v_out_hbm.at[pl.ds(0, C)], osems[p]).wait()
            pltpu.make_async_copy(
                nss[p], ns_out_hbm.at[pl.ds(0, C * D)], osems[p]).wait()

        pltpu.sync_copy(v_pos_hbm.at[pl.ds(wid * BW, BW)], v_idx_v)
        pltpu.sync_copy(neg_hbm.at[pl.ds(wid * BW * N, BW * N)], neg_idx_v)

        def fire(ci, p):
            pltpu.async_copy(
                v_w_hbm.at[v_idx_v.at[pl.ds(ci * C, C)]], vrs[p], gsems[p])
            for j in range(N):
                pltpu.async_copy(
                    v_w_hbm.at[neg_idx_v.at[pl.ds((ci * N + j) * C, C)]],
                    nrs[p].at[pl.ds(j * C, C)], gsems[p])

        def process(ci, p, first):
            wait_gathers(p)

            @pl.when(jnp.logical_not(first))
            def _():
                wait_outs(p)
            neg_rows_v, out_ns_v = nrs[p], nss[p]

            def b_body(b, carry2):
                rows = [neg_rows_v[b * N + nn] for nn in range(N)]
                while len(rows) > 1:
                    rows = [rows[i] + rows[i + 1] for i in range(0, len(rows) - 1, 2)] + (
                        [rows[-1]] if len(rows) % 2 else [])
                out_ns_v[pl.ds(b * D, D)] = rows[0]
                return carry2
            lax.fori_loop(0, C, b_body, 0, unroll=2)
            base = wid * BW + ci * C
            pltpu.async_copy(vrs[p], v_out_hbm.at[pl.ds(base, C)], osems[p])
            pltpu.async_copy(out_ns_v, ns_out_hbm.at[pl.ds(base * D, C * D)],
                             osems[p])

        fire(0, 0)

        def pair_body(kk, carry):
            ka = 2 * kk
            fire(ka + 1, 1)
            process(ka, 0, kk == 0)

            @pl.when(kk < n_chunks // 2 - 1)
            def _():
                fire(ka + 2, 0)
            process(ka + 1, 1, kk == 0)
            return carry
        lax.fori_loop(0, n_chunks // 2, pair_body, 0)
        wait_outs(0)
        wait_outs(1)

    return sc_kernel(v_pos, neg_idx, v_rm)


def _tc_loss_body(u_ref, v_ref, ns_ref, out_ref, *, D, inv_b):
    xu = u_ref[...]
    xp = xu * v_ref[...]
    xn = xu * ns_ref[...]
    lanes = xp.shape[-1]
    g = lanes // D
    i0 = lax.broadcasted_iota(jnp.int32, (lanes, g), 0)
    i1 = lax.broadcasted_iota(jnp.int32, (lanes, g), 1)
    s_mat = (i0 // D == i1).astype(jnp.float32)
    ps = jnp.dot(xp, s_mat, preferred_element_type=jnp.float32)
    ns = jnp.dot(xn, s_mat, preferred_element_type=jnp.float32)

    def log_sigmoid(x):
        return jnp.minimum(x, 0.0) - jnp.log1p(jnp.exp(-jnp.abs(x)))

    loss = log_sigmoid(ps) + log_sigmoid(ns)
    out_ref[0, 0] = -jnp.sum(loss) * inv_b


def kernel(u_pos, v_pos, v_neg, batch_size, u_weight, v_weight):
    B = u_pos.shape[0]
    V, D = u_weight.shape
    N = v_neg.shape[-1]
    info = plsc.get_sparse_core_info()
    NC, NS = info.num_cores, info.num_subcores
    NW = NC * NS

    v_lin = _sc_row_major(v_weight.T, V=V, D=D, NC=NC, NW=NW)

    neg_idx = v_neg.reshape(-1).astype(jnp.int32)
    embed_v, neg_sum = _sc_partials(
        v_pos.astype(jnp.int32), neg_idx, v_lin.reshape(V, D),
        B=B, V=V, D=D, N=N, NC=NC, NW=NW)

    embed_u = jnp.take(u_weight, u_pos, axis=0)

    lanes = 128
    rows = B * D // lanes
    u2d = embed_u.reshape(rows, lanes)
    v2d = embed_v.reshape(rows, lanes)
    ns2d = neg_sum.reshape(rows, lanes)

    out = pl.pallas_call(
        functools.partial(_tc_loss_body, D=D, inv_b=1.0 / B),
        out_shape=jax.ShapeDtypeStruct((1, 1), jnp.float32),
        out_specs=pl.BlockSpec(memory_space=pltpu.SMEM),
    )(u2d, v2d, ns2d)
    return out.reshape(())
